# baseline (speedup 1.0000x reference)
import functools

import jax
import jax.numpy as jnp
from jax import lax
from jax.experimental import pallas as pl
from jax.experimental.pallas import tpu as pltpu
from jax.experimental.pallas import tpu_sc as plsc

N_NODES = 10000
D_FEAT = 128
N_EDGES = 320000

NUM_CORES = 2
NUM_SUBCORES = 16
NUM_WORKERS = NUM_CORES * NUM_SUBCORES
CHUNK = 64
CPW = 160
SSUB = 32
EDGES_PER_WORKER = CPW * CHUNK
E_PAD = NUM_WORKERS * EDGES_PER_WORKER
N_PAD = 10000
ROWS_PER_TILE = N_PAD // NUM_SUBCORES
IDXSUB = 8


@functools.partial(
    pl.kernel,
    out_type=jax.ShapeDtypeStruct((NUM_CORES * N_PAD, D_FEAT), jnp.float32),
    mesh=plsc.VectorSubcoreMesh(core_axis_name="c", subcore_axis_name="s"),
    compiler_params=pltpu.CompilerParams(use_tc_tiling_on_sc=False),
    scratch_types=[
        pltpu.VMEM_SHARED((N_PAD, D_FEAT), jnp.float32),
        pltpu.VMEM((CPW, CHUNK), jnp.int32),
        pltpu.VMEM((CHUNK // SSUB, SSUB), jnp.int32),
        pltpu.VMEM((CHUNK // SSUB, SSUB), jnp.int32),
        pltpu.VMEM((CHUNK // SSUB, SSUB), jnp.int32),
        pltpu.VMEM((CHUNK // SSUB, SSUB), jnp.int32),
        pltpu.VMEM((CHUNK,), jnp.float32),
        pltpu.VMEM((CHUNK,), jnp.float32),
        pltpu.VMEM((CHUNK, D_FEAT), jnp.float32),
        pltpu.VMEM((CHUNK, D_FEAT), jnp.float32),
        pltpu.VMEM((CHUNK, D_FEAT), jnp.float32),
        pltpu.VMEM((CHUNK, D_FEAT), jnp.float32),
        pltpu.SemaphoreType.DMA,
        pltpu.SemaphoreType.DMA,
        pltpu.SemaphoreType.DMA,
        pltpu.SemaphoreType.DMA,
    ],
)
def _conv_pass(x_hbm, src_hbm, dst_hbm, w_hbm, out_hbm,
               acc, src_v, d0, d1, d2, d3, w0, w1, g0, g1, s0, s1,
               gsem0, gsem1, ssem0, ssem1):
    c = lax.axis_index("c")
    s = lax.axis_index("s")
    wid = c * NUM_SUBCORES + s
    gbuf = (g0, g1)
    sbuf = (s0, s1)
    dbuf = (d0, d1, d2, d3)
    wbuf = (w0, w1)
    gsem = (gsem0, gsem1)
    ssem = (ssem0, ssem1)

    ebase = wid * EDGES_PER_WORKER
    cbase = wid * CPW

    def load_idx(k, _):
        pltpu.sync_copy(src_hbm.at[pl.ds(cbase + k * IDXSUB, IDXSUB)],
                        src_v.at[pl.ds(k * IDXSUB, IDXSUB)])
        return 0

    lax.fori_loop(0, CPW // IDXSUB, load_idx, 0)

    def start_fetch(ci, p, q):
        pltpu.async_copy(x_hbm.at[src_v.at[ci]], gbuf[p], gsem[p])
        pltpu.async_copy(w_hbm.at[pl.ds(ebase + ci * CHUNK, CHUNK)],
                         wbuf[p], gsem[p])
        pltpu.async_copy(
            dst_hbm.at[pl.ds((ebase + ci * CHUNK) // SSUB, CHUNK // SSUB)],
            dbuf[q], gsem[p])

    def wait_fetch(ci, p, q):
        pltpu.make_async_copy(x_hbm.at[src_v.at[ci]], gbuf[p], gsem[p]).wait()
        pltpu.make_async_copy(w_hbm.at[pl.ds(ebase + ci * CHUNK, CHUNK)],
                              wbuf[p], gsem[p]).wait()
        pltpu.make_async_copy(
            dst_hbm.at[pl.ds((ebase + ci * CHUNK) // SSUB, CHUNK // SSUB)],
            dbuf[q], gsem[p]).wait()

    def start_scatter(ci, p, q):
        for t in range(CHUNK // SSUB):
            pltpu.async_copy(sbuf[p].at[pl.ds(t * SSUB, SSUB)],
                             acc.at[dbuf[q].at[t]], ssem[p], add=True)

    def wait_scatter(ci, p, q):
        for t in range(CHUNK // SSUB):
            pltpu.make_async_copy(sbuf[p].at[pl.ds(t * SSUB, SSUB)],
                                  acc.at[dbuf[q].at[t]], ssem[p]).wait()

    start_fetch(0, 0, 0)
    start_fetch(1, 1, 1)

    zvec = jnp.zeros((16,), jnp.float32)

    def zero_rows(i, _):
        for j in range(D_FEAT // 16):
            s0[i, pl.ds(j * 16, 16)] = zvec
        return 0

    lax.fori_loop(0, CHUNK, zero_rows, 0)
    row0 = s * ROWS_PER_TILE
    for k in range(ROWS_PER_TILE // CHUNK):
        pltpu.sync_copy(s0, acc.at[pl.ds(row0 + k * CHUNK, CHUNK)])
    zrem = ROWS_PER_TILE % CHUNK
    if zrem:
        pltpu.sync_copy(s0.at[pl.ds(0, zrem)],
                        acc.at[pl.ds(row0 + (ROWS_PER_TILE // CHUNK) * CHUNK, zrem)])
    plsc.subcore_barrier()

    def scale(ci, p):
        g, sb, wv = gbuf[p], sbuf[p], wbuf[p]

        def scale_group(gi, _):
            wvec = wv[pl.ds(gi * 16, 16)]
            for l in range(16):
                e = gi * 16 + l
                wsp = wvec[l]
                for j in range(D_FEAT // 16):
                    sl = pl.ds(j * 16, 16)
                    sb[e, sl] = g[e, sl] * wsp
            return 0

        lax.fori_loop(0, CHUNK // 16, scale_group, 0)

    def quad_body(k, _):
        for j in range(4):
            ci = 4 * k + j
            p = j % 2
            wait_fetch(ci, p, j)
            if j >= 2:
                wait_scatter(ci - 2, p, j - 2)
            else:
                @pl.when(k > 0)
                def _():
                    wait_scatter(ci - 2, p, j + 2)
            scale(ci, p)
            if j < 2:
                start_fetch(ci + 2, p, (j + 2) % 4)
            else:
                @pl.when(k < CPW // 4 - 1)
                def _():
                    start_fetch(ci + 2, p, (j + 2) % 4)
            start_scatter(ci, p, j)
        return 0

    lax.fori_loop(0, CPW // 4, quad_body, 0)

    wait_scatter(CPW - 2, 0, 2)
    wait_scatter(CPW - 1, 1, 3)
    plsc.subcore_barrier()

    out0 = c * N_PAD + row0
    pltpu.sync_copy(acc.at[pl.ds(row0, ROWS_PER_TILE)],
                    out_hbm.at[pl.ds(out0, ROWS_PER_TILE)])


def _add_body(a_ref, b_ref, o_ref):
    o_ref[...] = a_ref[...] + b_ref[...]


def _add_relu_body(a_ref, b_ref, o_ref):
    o_ref[...] = jnp.maximum(a_ref[...] + b_ref[...], 0.0)


def _combine(parts, relu):
    body = _add_relu_body if relu else _add_body
    blk = 1000
    nblk = N_PAD // blk
    return pl.pallas_call(
        body,
        grid=(nblk,),
        in_specs=[pl.BlockSpec((blk, D_FEAT), lambda i: (i, 0)),
                  pl.BlockSpec((blk, D_FEAT), lambda i, n=nblk: (i + n, 0))],
        out_specs=pl.BlockSpec((blk, D_FEAT), lambda i: (i, 0)),
        out_shape=jax.ShapeDtypeStruct((N_PAD, D_FEAT), jnp.float32),
    )(parts, parts)


def kernel(x, edge_index, edge_weight):
    src = edge_index[0].astype(jnp.int32)
    dst = edge_index[1].astype(jnp.int32)
    w = edge_weight.astype(jnp.float32)
    pad = E_PAD - N_EDGES
    spread = jnp.arange(pad, dtype=jnp.int32) % N_NODES
    src = jnp.concatenate([src, spread]).reshape(NUM_WORKERS * CPW, CHUNK)
    dst = jnp.concatenate([dst, spread]).reshape(E_PAD // SSUB, SSUB)
    w = jnp.concatenate([w, jnp.zeros((pad,), jnp.float32)])

    p = _conv_pass(x, src, dst, w)
    h = _combine(p, relu=False)
    p2 = _conv_pass(h, src, dst, w)
    return _combine(p2, relu=True)


# speedup vs baseline: 1.5905x; 1.5905x over previous
import functools

import jax
import jax.numpy as jnp
from jax import lax
from jax.experimental import pallas as pl
from jax.experimental.pallas import tpu as pltpu
from jax.experimental.pallas import tpu_sc as plsc

N_NODES = 10000
D_FEAT = 128
N_EDGES = 320000

NUM_CORES = 2
NUM_SUBCORES = 16
NUM_WORKERS = NUM_CORES * NUM_SUBCORES
CHUNK = 32
CPW = 320
EDGES_PER_WORKER = CPW * CHUNK
E_PAD = NUM_WORKERS * EDGES_PER_WORKER
N_PAD = 10000
ROWS_PER_TILE = N_PAD // NUM_SUBCORES
IDXSUB = 8
UNROLL = 8


@functools.partial(
    pl.kernel,
    out_type=jax.ShapeDtypeStruct((NUM_CORES * N_PAD, D_FEAT), jnp.float32),
    mesh=plsc.VectorSubcoreMesh(core_axis_name="c", subcore_axis_name="s"),
    compiler_params=pltpu.CompilerParams(use_tc_tiling_on_sc=False),
    scratch_types=[
        pltpu.VMEM_SHARED((N_PAD, D_FEAT), jnp.float32),
        pltpu.VMEM((CPW, CHUNK), jnp.int32),
        pltpu.VMEM((UNROLL, CHUNK), jnp.int32),
        pltpu.VMEM((4, CHUNK), jnp.float32),
        pltpu.VMEM((CHUNK, D_FEAT), jnp.float32),
        pltpu.VMEM((CHUNK, D_FEAT), jnp.float32),
        pltpu.VMEM((CHUNK, D_FEAT), jnp.float32),
        pltpu.VMEM((CHUNK, D_FEAT), jnp.float32),
        pltpu.VMEM((CHUNK, D_FEAT), jnp.float32),
        pltpu.VMEM((CHUNK, D_FEAT), jnp.float32),
        pltpu.SemaphoreType.DMA,
        pltpu.SemaphoreType.DMA,
        pltpu.SemaphoreType.DMA,
        pltpu.SemaphoreType.DMA,
        pltpu.SemaphoreType.DMA,
        pltpu.SemaphoreType.DMA,
    ],
)
def _conv_pass(x_hbm, src_hbm, dst_hbm, w_hbm, out_hbm,
               acc, src_v, dst_v, w_v, g0, g1, g2, g3, s0, s1,
               gsem0, gsem1, gsem2, gsem3, ssem0, ssem1):
    c = lax.axis_index("c")
    s = lax.axis_index("s")
    wid = c * NUM_SUBCORES + s
    gbuf = (g0, g1, g2, g3)
    sbuf = (s0, s1)
    gsem = (gsem0, gsem1, gsem2, gsem3)
    ssem = (ssem0, ssem1)

    ebase = wid * EDGES_PER_WORKER
    cbase = wid * CPW

    def load_idx(k, _):
        pltpu.sync_copy(src_hbm.at[pl.ds(cbase + k * IDXSUB, IDXSUB)],
                        src_v.at[pl.ds(k * IDXSUB, IDXSUB)])
        return 0

    lax.fori_loop(0, CPW // IDXSUB, load_idx, 0)

    def start_fetch(ci, pg, q):
        pltpu.async_copy(x_hbm.at[src_v.at[ci]], gbuf[pg], gsem[pg])
        pltpu.async_copy(w_hbm.at[pl.ds(ebase + ci * CHUNK, CHUNK)],
                         w_v.at[pg], gsem[pg])
        pltpu.async_copy(dst_hbm.at[pl.ds(ebase + ci * CHUNK, CHUNK)],
                         dst_v.at[q], gsem[pg])

    def wait_fetch(ci, pg, q):
        pltpu.make_async_copy(x_hbm.at[src_v.at[ci]], gbuf[pg], gsem[pg]).wait()
        pltpu.make_async_copy(w_hbm.at[pl.ds(ebase + ci * CHUNK, CHUNK)],
                              w_v.at[pg], gsem[pg]).wait()
        pltpu.make_async_copy(dst_hbm.at[pl.ds(ebase + ci * CHUNK, CHUNK)],
                              dst_v.at[q], gsem[pg]).wait()

    def start_scatter(ci, ps, q):
        pltpu.async_copy(sbuf[ps], acc.at[dst_v.at[q]], ssem[ps], add=True)

    def wait_scatter(ci, ps, q):
        pltpu.make_async_copy(sbuf[ps], acc.at[dst_v.at[q]], ssem[ps]).wait()

    for ci in range(4):
        start_fetch(ci, ci, ci)

    zvec = jnp.zeros((16,), jnp.float32)

    def zero_rows(i, _):
        for j in range(D_FEAT // 16):
            s0[i, pl.ds(j * 16, 16)] = zvec
        return 0

    lax.fori_loop(0, CHUNK, zero_rows, 0)
    row0 = s * ROWS_PER_TILE
    for k in range(ROWS_PER_TILE // CHUNK):
        pltpu.sync_copy(s0, acc.at[pl.ds(row0 + k * CHUNK, CHUNK)])
    zrem = ROWS_PER_TILE % CHUNK
    if zrem:
        pltpu.sync_copy(s0.at[pl.ds(0, zrem)],
                        acc.at[pl.ds(row0 + (ROWS_PER_TILE // CHUNK) * CHUNK, zrem)])
    plsc.subcore_barrier()

    def scale(ci, pg, ps):
        g, sb = gbuf[pg], sbuf[ps]

        def scale_group(gi, _):
            wvec = w_v[pg, pl.ds(gi * 16, 16)]
            for l in range(16):
                e = gi * 16 + l
                wsp = wvec[l]
                for j in range(D_FEAT // 16):
                    sl = pl.ds(j * 16, 16)
                    sb[e, sl] = g[e, sl] * wsp
            return 0

        lax.fori_loop(0, CHUNK // 16, scale_group, 0)

    def oct_body(k, _):
        for j in range(UNROLL):
            ci = UNROLL * k + j
            pg = j % 4
            ps = j % 2
            wait_fetch(ci, pg, j)
            if j >= 2:
                wait_scatter(ci - 2, ps, j - 2)
            else:
                @pl.when(k > 0)
                def _():
                    wait_scatter(ci - 2, ps, j + UNROLL - 2)
            scale(ci, pg, ps)
            if j < 4:
                start_fetch(ci + 4, pg, (j + 4) % UNROLL)
            else:
                @pl.when(k < CPW // UNROLL - 1)
                def _():
                    start_fetch(ci + 4, pg, (j + 4) % UNROLL)
            start_scatter(ci, ps, j)
        return 0

    lax.fori_loop(0, CPW // UNROLL, oct_body, 0)

    wait_scatter(CPW - 2, 0, UNROLL - 2)
    wait_scatter(CPW - 1, 1, UNROLL - 1)
    plsc.subcore_barrier()

    out0 = c * N_PAD + row0
    pltpu.sync_copy(acc.at[pl.ds(row0, ROWS_PER_TILE)],
                    out_hbm.at[pl.ds(out0, ROWS_PER_TILE)])


def _add_body(a_ref, b_ref, o_ref):
    o_ref[...] = a_ref[...] + b_ref[...]


def _add_relu_body(a_ref, b_ref, o_ref):
    o_ref[...] = jnp.maximum(a_ref[...] + b_ref[...], 0.0)


def _combine(parts, relu):
    body = _add_relu_body if relu else _add_body
    blk = 1000
    nblk = N_PAD // blk
    return pl.pallas_call(
        body,
        grid=(nblk,),
        in_specs=[pl.BlockSpec((blk, D_FEAT), lambda i: (i, 0)),
                  pl.BlockSpec((blk, D_FEAT), lambda i, n=nblk: (i + n, 0))],
        out_specs=pl.BlockSpec((blk, D_FEAT), lambda i: (i, 0)),
        out_shape=jax.ShapeDtypeStruct((N_PAD, D_FEAT), jnp.float32),
    )(parts, parts)


def kernel(x, edge_index, edge_weight):
    src = edge_index[0].astype(jnp.int32)
    dst = edge_index[1].astype(jnp.int32)
    w = edge_weight.astype(jnp.float32)
    pad = E_PAD - N_EDGES
    spread = jnp.arange(pad, dtype=jnp.int32) % N_NODES
    src = jnp.concatenate([src, spread]).reshape(NUM_WORKERS * CPW, CHUNK)
    dst = jnp.concatenate([dst, spread])
    w = jnp.concatenate([w, jnp.zeros((pad,), jnp.float32)])

    p = _conv_pass(x, src, dst, w)
    h = _combine(p, relu=False)
    p2 = _conv_pass(h, src, dst, w)
    return _combine(p2, relu=True)
